# Initial kernel scaffold; baseline (speedup 1.0000x reference)
import jax
import jax.numpy as jnp
from jax.experimental import pallas as pl


def kernel(freq, edge_index, edge_weight, Wl0, bl0, Wr0, Wl1, bl1, Wr1, Wl2, bl2, Wr2, pool_w, W1, b1, W2, b2):
    raise NotImplementedError("write your pallas kernel here")


# speedup vs baseline: 3.4811x; 3.4811x over previous
import functools

import jax
import jax.numpy as jnp
from jax import lax
from jax.experimental import pallas as pl
from jax.experimental.pallas import tpu as pltpu
from jax.experimental.pallas import tpu_sc as plsc


_G = 80
_CW = 16


def _sc_agg_body(with_cnt, B, N, E, D, half,
                 xl_hbm, xr_hbm, srcg_hbm, dstl_hbm, *refs):
    if with_cnt:
        (aggl_hbm, aggr_hbm, cnt_hbm, s1d, d1d, rows, zbuf, ones, czero,
         acc_sh, cnt_sh, gsem) = refs
    else:
        (aggl_hbm, aggr_hbm, s1d, d1d, rows, zbuf, acc_sh, gsem) = refs
        cnt_sh = czero = ones = None
    c = lax.axis_index("c")
    s = lax.axis_index("s")

    n_sub = 16
    epw = E // n_sub
    n_chunk = epw // _G
    rpw = (N // n_sub) // 8 * 8
    tail = N - n_sub * rpw
    zrows = zbuf.shape[0]
    n_zcopy = rpw // zrows

    @pl.loop(0, zrows * (half // 16))
    def _(i):
        r = i // (half // 16)
        col = (i % (half // 16)) * 16
        zbuf[r, pl.ds(col, 16)] = jnp.zeros((16,), jnp.float32)

    if with_cnt:
        @pl.loop(0, _G)
        def _(i):
            ones[i, :] = jnp.ones((_CW,), jnp.float32)

        @pl.loop(0, czero.shape[0])
        def _(i):
            czero[i, :] = jnp.zeros((_CW,), jnp.float32)

    for bi in range(2):
        b = 2 * c + bi
        for h, (xh_hbm, aggh_hbm) in enumerate(
                ((xl_hbm, aggl_hbm), (xr_hbm, aggr_hbm))):
            count_now = with_cnt and h == 0
            for z in range(n_zcopy):
                pltpu.sync_copy(zbuf,
                                acc_sh.at[pl.ds(s * rpw + z * zrows, zrows)])
            if count_now:
                pltpu.sync_copy(czero, cnt_sh.at[pl.ds(s * rpw, rpw)])

            @pl.when(s == n_sub - 1)
            def _():
                pltpu.sync_copy(zbuf.at[pl.ds(0, tail)],
                                acc_sh.at[pl.ds(n_sub * rpw, tail)])
                if count_now:
                    pltpu.sync_copy(czero.at[pl.ds(0, tail)],
                                    cnt_sh.at[pl.ds(n_sub * rpw, tail)])

            plsc.subcore_barrier()

            base = b * E + s * epw

            @pl.loop(0, n_chunk)
            def _(i):
                off = base + i * _G
                pltpu.sync_copy(srcg_hbm.at[pl.ds(off, _G)], s1d)
                pltpu.sync_copy(dstl_hbm.at[pl.ds(off, _G)], d1d)
                pltpu.async_copy(xh_hbm.at[s1d], rows, gsem).wait()
                pltpu.sync_copy(rows, acc_sh.at[d1d], add=True)
                if count_now:
                    pltpu.sync_copy(ones, cnt_sh.at[d1d], add=True)

            plsc.subcore_barrier()

            r0 = s * rpw
            pltpu.sync_copy(acc_sh.at[pl.ds(r0, rpw)],
                            aggh_hbm.at[pl.ds(b * N + r0, rpw)])
            if count_now:
                pltpu.sync_copy(cnt_sh.at[pl.ds(r0, rpw)],
                                cnt_hbm.at[pl.ds(b * N + r0, rpw)])

            @pl.when(s == n_sub - 1)
            def _():
                pltpu.sync_copy(acc_sh.at[pl.ds(n_sub * rpw, tail)],
                                aggh_hbm.at[pl.ds(b * N + n_sub * rpw, tail)])
                if count_now:
                    pltpu.sync_copy(cnt_sh.at[pl.ds(n_sub * rpw, tail)],
                                    cnt_hbm.at[pl.ds(b * N + n_sub * rpw, tail)])


def _make_sc_agg(with_cnt, B, N, E, D):
    half = D // 2
    mesh = plsc.VectorSubcoreMesh(core_axis_name="c", subcore_axis_name="s")
    out_type = [jax.ShapeDtypeStruct((B * N, half), jnp.float32),
                jax.ShapeDtypeStruct((B * N, half), jnp.float32)]
    scratch = [
        pltpu.VMEM((_G,), jnp.int32),
        pltpu.VMEM((_G,), jnp.int32),
        pltpu.VMEM((_G, half), jnp.float32),
        pltpu.VMEM((104, half), jnp.float32),
    ]
    if with_cnt:
        out_type.append(jax.ShapeDtypeStruct((B * N, _CW), jnp.float32))
        scratch.append(pltpu.VMEM((_G, _CW), jnp.float32))
        scratch.append(pltpu.VMEM((624, _CW), jnp.float32))
        scratch.append(pltpu.VMEM_SHARED((N, half), jnp.float32))
        scratch.append(pltpu.VMEM_SHARED((N, _CW), jnp.float32))
    else:
        scratch.append(pltpu.VMEM_SHARED((N, half), jnp.float32))
    scratch.append(pltpu.SemaphoreType.DMA)

    return pl.kernel(
        functools.partial(_sc_agg_body, with_cnt, B, N, E, D, half),
        out_type=tuple(out_type),
        mesh=mesh,
        scratch_types=tuple(scratch),
        compiler_params=pltpu.CompilerParams(use_tc_tiling_on_sc=False),
    )


def _tc_layer_body(relu, half, al_ref, ar_ref, cnt_ref, xl_ref, xr_ref,
                   wl_ref, wr_ref, b_ref, ol_ref, or_ref):
    inv = 1.0 / jnp.maximum(cnt_ref[:, 0:1], 1.0)
    dot = functools.partial(jnp.dot, preferred_element_type=jnp.float32)
    out = (dot(al_ref[...] * inv, wl_ref[0:half, :])
           + dot(ar_ref[...] * inv, wl_ref[half:, :])
           + dot(xl_ref[...], wr_ref[0:half, :])
           + dot(xr_ref[...], wr_ref[half:, :])
           + b_ref[...][None, :])
    if relu:
        out = jnp.maximum(out, 0.0)
    ol_ref[...] = out[:, 0:half]
    or_ref[...] = out[:, half:]


def _tc_layer(al, ar, cnt, xl, xr, wl, wr, b, relu):
    M, half = xl.shape
    D = 2 * half
    H = wl.shape[1]
    BR = 1000
    return pl.pallas_call(
        functools.partial(_tc_layer_body, relu, half),
        grid=(M // BR,),
        in_specs=[
            pl.BlockSpec((BR, half), lambda i: (i, 0)),
            pl.BlockSpec((BR, half), lambda i: (i, 0)),
            pl.BlockSpec((BR, _CW), lambda i: (i, 0)),
            pl.BlockSpec((BR, half), lambda i: (i, 0)),
            pl.BlockSpec((BR, half), lambda i: (i, 0)),
            pl.BlockSpec((D, H), lambda i: (0, 0)),
            pl.BlockSpec((D, H), lambda i: (0, 0)),
            pl.BlockSpec((H,), lambda i: (0,)),
        ],
        out_specs=[pl.BlockSpec((BR, H // 2), lambda i: (i, 0)),
                   pl.BlockSpec((BR, H // 2), lambda i: (i, 0))],
        out_shape=[jax.ShapeDtypeStruct((M, H // 2), jnp.float32),
                   jax.ShapeDtypeStruct((M, H // 2), jnp.float32)],
    )(al, ar, cnt, xl, xr, wl, wr, b)


def _tc_score_body(half, xl_ref, xr_ref, pw_ref, s_ref):
    pw = pw_ref[...]
    pwn = pw / jnp.sqrt(jnp.sum(pw * pw))
    dg = functools.partial(lax.dot_general,
                           preferred_element_type=jnp.float32)
    s_ref[0] = jnp.tanh(
        dg(xl_ref[...], pwn[0:half], (((2,), (0,)), ((), ())))
        + dg(xr_ref[...], pwn[half:], (((2,), (0,)), ((), ()))))


def _tc_select_body(k, s_ref, w_ref):
    score = s_ref[...]
    NBn, Bb, BNn = score.shape

    bits = lax.bitcast_convert_type(score, jnp.uint32)
    key = jnp.where(bits >= jnp.uint32(0x80000000),
                    ~bits, bits | jnp.uint32(0x80000000))

    def bit_step(i, pref):
        cand = pref | (jnp.uint32(1) << (jnp.uint32(31) - jnp.uint32(i)))
        cnt = jnp.sum((key >= cand).astype(jnp.int32), axis=(0, 2),
                      keepdims=True)
        return jnp.where(cnt >= k, cand, pref)

    tau = lax.fori_loop(0, 32, bit_step,
                        jnp.zeros((1, Bb, 1), jnp.uint32))

    gt = key > tau
    eq = key == tau
    m = k - jnp.sum(gt.astype(jnp.int32), axis=(0, 2), keepdims=True)

    idx = (lax.broadcasted_iota(jnp.int32, (NBn, Bb, BNn), 0) * BNn
           + lax.broadcasted_iota(jnp.int32, (NBn, Bb, BNn), 2))

    def cut_step(i, pref):
        cand = pref + (jnp.int32(1) << (14 - i))
        cnt = jnp.sum((eq & (idx < cand)).astype(jnp.int32), axis=(0, 2),
                      keepdims=True)
        return jnp.where(cnt <= m, cand, pref)

    cut = lax.fori_loop(0, 15, cut_step, jnp.zeros((1, Bb, 1), jnp.int32))

    sel = gt | (eq & (idx < cut))
    w_ref[...] = jnp.where(sel, score, 0.0) * (1.0 / k)


def _tc_paccum_body(w_ref, xl_ref, xr_ref, pl_ref, pr_ref):
    i = pl.program_id(0)

    @pl.when(i == 0)
    def _():
        pl_ref[...] = jnp.zeros_like(pl_ref)
        pr_ref[...] = jnp.zeros_like(pr_ref)

    dg = functools.partial(lax.dot_general,
                           preferred_element_type=jnp.float32)
    w = w_ref[0]
    pl_ref[...] += dg(w, xl_ref[...], (((1,), (1,)), ((0,), (0,))))
    pr_ref[...] += dg(w, xr_ref[...], (((1,), (1,)), ((0,), (0,))))


def _tc_mlp_body(half, pl_ref, pr_ref, w1_ref, b1_ref, w2_ref, b2_ref, o_ref):
    dot = functools.partial(jnp.dot, preferred_element_type=jnp.float32)
    h = jnp.maximum(dot(pl_ref[...], w1_ref[0:half, :])
                    + dot(pr_ref[...], w1_ref[half:, :])
                    + b1_ref[...][None, :], 0.0)
    o_ref[...] = dot(h, w2_ref[...]) + b2_ref[...][None, :]


def _tc_pool(x3l, x3r, pool_w, W1, b1, W2, b2):
    Bb, Nn, half = x3l.shape
    k = -(-Nn // 2)
    D = 2 * half
    MH = W1.shape[1]
    C = W2.shape[1]
    BN = 2000
    NB = Nn // BN

    score = pl.pallas_call(
        functools.partial(_tc_score_body, half),
        grid=(NB,),
        in_specs=[
            pl.BlockSpec((Bb, BN, half), lambda i: (0, i, 0)),
            pl.BlockSpec((Bb, BN, half), lambda i: (0, i, 0)),
            pl.BlockSpec((D,), lambda i: (0,)),
        ],
        out_specs=pl.BlockSpec((1, Bb, BN), lambda i: (i, 0, 0)),
        out_shape=jax.ShapeDtypeStruct((NB, Bb, BN), jnp.float32),
    )(x3l, x3r, pool_w)

    w = pl.pallas_call(
        functools.partial(_tc_select_body, k),
        out_shape=jax.ShapeDtypeStruct((NB, Bb, BN), jnp.float32),
    )(score)

    pooled_l, pooled_r = pl.pallas_call(
        _tc_paccum_body,
        grid=(NB,),
        in_specs=[
            pl.BlockSpec((1, Bb, BN), lambda i: (i, 0, 0)),
            pl.BlockSpec((Bb, BN, half), lambda i: (0, i, 0)),
            pl.BlockSpec((Bb, BN, half), lambda i: (0, i, 0)),
        ],
        out_specs=[pl.BlockSpec((Bb, half), lambda i: (0, 0)),
                   pl.BlockSpec((Bb, half), lambda i: (0, 0))],
        out_shape=[jax.ShapeDtypeStruct((Bb, half), jnp.float32),
                   jax.ShapeDtypeStruct((Bb, half), jnp.float32)],
    )(w, x3l, x3r)

    return pl.pallas_call(
        functools.partial(_tc_mlp_body, half),
        out_shape=jax.ShapeDtypeStruct((Bb, C), jnp.float32),
    )(pooled_l, pooled_r, W1, b1, W2, b2)


def kernel(freq, edge_index, edge_weight, Wl0, bl0, Wr0, Wl1, bl1, Wr1,
           Wl2, bl2, Wr2, pool_w, W1, b1, W2, b2):
    B, N, D = freq.shape
    E = edge_index.shape[2]
    half = D // 2

    x0 = freq.reshape(B * N, D)
    x0l = x0[:, 0:half]
    x0r = x0[:, half:]
    offs = (jnp.arange(B, dtype=edge_index.dtype) * N)[:, None]
    srcg = (edge_index[:, 0, :] + offs).reshape(-1)
    dstl = edge_index[:, 1, :].reshape(-1)

    agg_cnt = _make_sc_agg(True, B, N, E, D)
    agg_only = _make_sc_agg(False, B, N, E, D)

    a0l, a0r, cnt = agg_cnt(x0l, x0r, srcg, dstl)
    x1l, x1r = _tc_layer(a0l, a0r, cnt, x0l, x0r, Wl0, Wr0, bl0, relu=True)
    a1l, a1r = agg_only(x1l, x1r, srcg, dstl)
    x2l, x2r = _tc_layer(a1l, a1r, cnt, x1l, x1r, Wl1, Wr1, bl1, relu=True)
    a2l, a2r = agg_only(x2l, x2r, srcg, dstl)
    x3l, x3r = _tc_layer(a2l, a2r, cnt, x2l, x2r, Wl2, Wr2, bl2, relu=False)

    return _tc_pool(x3l.reshape(B, N, half), x3r.reshape(B, N, half),
                    pool_w, W1, b1, W2, b2)


# speedup vs baseline: 4.5778x; 1.3151x over previous
import functools

import jax
import jax.numpy as jnp
from jax import lax
from jax.experimental import pallas as pl
from jax.experimental.pallas import tpu as pltpu
from jax.experimental.pallas import tpu_sc as plsc


_G = 80
_CW = 16


_NBUF = 4


def _sc_agg_body(with_cnt, B, N, E, D, half,
                 xl_hbm, xr_hbm, srcg_hbm, dstl_hbm, *refs):
    refs = list(refs)
    aggl_hbm = refs.pop(0)
    aggr_hbm = refs.pop(0)
    cnt_hbm = refs.pop(0) if with_cnt else None
    s2d = refs.pop(0)
    d2d = refs.pop(0)
    rows = [refs.pop(0) for _ in range(_NBUF)]
    zbuf = refs.pop(0)
    if with_cnt:
        ones = refs.pop(0)
        czero = refs.pop(0)
    acc_sh = refs.pop(0)
    cnt_sh = refs.pop(0) if with_cnt else None
    gsem = [refs.pop(0) for _ in range(_NBUF)]
    ssem = [refs.pop(0) for _ in range(_NBUF)]
    csem = [refs.pop(0) for _ in range(_NBUF)] if with_cnt else None
    assert not refs

    c = lax.axis_index("c")
    s = lax.axis_index("s")

    n_sub = 16
    n_chunk = srcg_hbm.shape[1]
    rpw = (N // n_sub) // 8 * 8
    tail = N - n_sub * rpw
    zrows = zbuf.shape[0]
    n_zcopy = rpw // zrows

    @pl.loop(0, zrows * (half // 16))
    def _(i):
        r = i // (half // 16)
        col = (i % (half // 16)) * 16
        zbuf[r, pl.ds(col, 16)] = jnp.zeros((16,), jnp.float32)

    if with_cnt:
        @pl.loop(0, _G)
        def _(i):
            ones[i, :] = jnp.ones((_CW,), jnp.float32)

        @pl.loop(0, czero.shape[0])
        def _(i):
            czero[i, :] = jnp.zeros((_CW,), jnp.float32)

    for bi in range(2):
        b = 2 * c + bi
        idx_row = b * n_sub + s
        for h, (xh_hbm, aggh_hbm) in enumerate(
                ((xl_hbm, aggl_hbm), (xr_hbm, aggr_hbm))):
            count_now = with_cnt and h == 0
            for z in range(n_zcopy):
                pltpu.sync_copy(zbuf,
                                acc_sh.at[pl.ds(s * rpw + z * zrows, zrows)])
            if count_now:
                pltpu.sync_copy(czero, cnt_sh.at[pl.ds(s * rpw, rpw)])

            @pl.when(s == n_sub - 1)
            def _():
                pltpu.sync_copy(zbuf.at[pl.ds(0, tail)],
                                acc_sh.at[pl.ds(n_sub * rpw, tail)])
                if count_now:
                    pltpu.sync_copy(czero.at[pl.ds(0, tail)],
                                    cnt_sh.at[pl.ds(n_sub * rpw, tail)])

            if h == 0:
                pltpu.sync_copy(srcg_hbm.at[idx_row], s2d)
                pltpu.sync_copy(dstl_hbm.at[idx_row], d2d)

            def fire_gather(ch, nb):
                pltpu.async_copy(xh_hbm.at[s2d.at[ch]], rows[nb], gsem[nb])

            def wait_gather(nb):
                pltpu.make_async_copy(xh_hbm.at[s2d.at[0]], rows[nb],
                                      gsem[nb]).wait()

            for nb in range(_NBUF):
                fire_gather(nb, nb)

            assert n_chunk % _NBUF == 0

            @pl.loop(0, n_chunk // _NBUF)
            def _(i):
                for nb in range(_NBUF):
                    ch = i * _NBUF + nb
                    wait_gather(nb)
                    pltpu.async_copy(rows[nb], acc_sh.at[d2d.at[ch]],
                                     ssem[nb], add=True)
                    if count_now:
                        pltpu.async_copy(ones, cnt_sh.at[d2d.at[ch]],
                                         csem[nb], add=True)
                    pltpu.make_async_copy(rows[nb], acc_sh.at[d2d.at[0]],
                                          ssem[nb]).wait()
                    if count_now:
                        pltpu.make_async_copy(ones, cnt_sh.at[d2d.at[0]],
                                              csem[nb]).wait()

                    @pl.when(ch + _NBUF < n_chunk)
                    def _():
                        fire_gather(ch + _NBUF, nb)

            r0 = s * rpw
            pltpu.sync_copy(acc_sh.at[pl.ds(r0, rpw)],
                            aggh_hbm.at[pl.ds(b * N + r0, rpw)])
            if count_now:
                pltpu.sync_copy(cnt_sh.at[pl.ds(r0, rpw)],
                                cnt_hbm.at[pl.ds(b * N + r0, rpw)])

            @pl.when(s == n_sub - 1)
            def _():
                pltpu.sync_copy(acc_sh.at[pl.ds(n_sub * rpw, tail)],
                                aggh_hbm.at[pl.ds(b * N + n_sub * rpw, tail)])
                if count_now:
                    pltpu.sync_copy(cnt_sh.at[pl.ds(n_sub * rpw, tail)],
                                    cnt_hbm.at[pl.ds(b * N + n_sub * rpw, tail)])


def _make_sc_agg(with_cnt, B, N, E, D, n_chunk):
    half = D // 2
    mesh = plsc.VectorSubcoreMesh(core_axis_name="c", subcore_axis_name="s")
    out_type = [jax.ShapeDtypeStruct((B * N, half), jnp.float32),
                jax.ShapeDtypeStruct((B * N, half), jnp.float32)]
    if with_cnt:
        out_type.append(jax.ShapeDtypeStruct((B * N, _CW), jnp.float32))
    scratch = [
        pltpu.VMEM((n_chunk, _G), jnp.int32),
        pltpu.VMEM((n_chunk, _G), jnp.int32),
    ]
    scratch += [pltpu.VMEM((_G, half), jnp.float32) for _ in range(_NBUF)]
    scratch.append(pltpu.VMEM((104, half), jnp.float32))
    if with_cnt:
        scratch.append(pltpu.VMEM((_G, _CW), jnp.float32))
        scratch.append(pltpu.VMEM((624, _CW), jnp.float32))
        scratch.append(pltpu.VMEM_SHARED((N + 8, half), jnp.float32))
        scratch.append(pltpu.VMEM_SHARED((N + 8, _CW), jnp.float32))
    else:
        scratch.append(pltpu.VMEM_SHARED((N + 8, half), jnp.float32))
    scratch += [pltpu.SemaphoreType.DMA] * (_NBUF * (3 if with_cnt else 2))

    return pl.kernel(
        functools.partial(_sc_agg_body, with_cnt, B, N, E, D, half),
        out_type=tuple(out_type),
        mesh=mesh,
        scratch_types=tuple(scratch),
        compiler_params=pltpu.CompilerParams(use_tc_tiling_on_sc=False),
    )


def _tc_layer_body(relu, half, al_ref, ar_ref, cnt_ref, xl_ref, xr_ref,
                   wl_ref, wr_ref, b_ref, ol_ref, or_ref):
    cnt = jnp.maximum(cnt_ref[:, 0:1], 1.0)
    mean = jnp.concatenate([al_ref[...], ar_ref[...]], axis=1) / cnt
    x = jnp.concatenate([xl_ref[...], xr_ref[...]], axis=1)
    dot = functools.partial(jnp.dot, preferred_element_type=jnp.float32)
    out = dot(mean, wl_ref[...]) + b_ref[...][None, :] + dot(x, wr_ref[...])
    if relu:
        out = jnp.maximum(out, 0.0)
    ol_ref[...] = out[:, 0:half]
    or_ref[...] = out[:, half:]


def _tc_layer(al, ar, cnt, xl, xr, wl, wr, b, relu):
    M, half = xl.shape
    D = 2 * half
    H = wl.shape[1]
    BR = 1000
    return pl.pallas_call(
        functools.partial(_tc_layer_body, relu, half),
        grid=(M // BR,),
        in_specs=[
            pl.BlockSpec((BR, half), lambda i: (i, 0)),
            pl.BlockSpec((BR, half), lambda i: (i, 0)),
            pl.BlockSpec((BR, _CW), lambda i: (i, 0)),
            pl.BlockSpec((BR, half), lambda i: (i, 0)),
            pl.BlockSpec((BR, half), lambda i: (i, 0)),
            pl.BlockSpec((D, H), lambda i: (0, 0)),
            pl.BlockSpec((D, H), lambda i: (0, 0)),
            pl.BlockSpec((H,), lambda i: (0,)),
        ],
        out_specs=[pl.BlockSpec((BR, H // 2), lambda i: (i, 0)),
                   pl.BlockSpec((BR, H // 2), lambda i: (i, 0))],
        out_shape=[jax.ShapeDtypeStruct((M, H // 2), jnp.float32),
                   jax.ShapeDtypeStruct((M, H // 2), jnp.float32)],
    )(al, ar, cnt, xl, xr, wl, wr, b)


def _tc_score_body(half, xl_ref, xr_ref, pw_ref, s_ref):
    pw = pw_ref[...]
    norm = jnp.sqrt(jnp.sum(pw * pw))
    x3 = jnp.concatenate([xl_ref[...], xr_ref[...]], axis=2)
    dg = functools.partial(lax.dot_general,
                           preferred_element_type=jnp.float32)
    s_ref[0] = jnp.tanh(dg(x3, pw, (((2,), (0,)), ((), ()))) / norm)


def _tc_select_body(k, s_ref, w_ref):
    score = s_ref[...]
    NBn, Bb, BNn = score.shape

    bits = lax.bitcast_convert_type(score, jnp.uint32)
    key = jnp.where(bits >= jnp.uint32(0x80000000),
                    ~bits, bits | jnp.uint32(0x80000000))

    def bit_step(i, pref):
        cand = pref | (jnp.uint32(1) << (jnp.uint32(31) - jnp.uint32(i)))
        cnt = jnp.sum((key >= cand).astype(jnp.int32), axis=(0, 2),
                      keepdims=True)
        return jnp.where(cnt >= k, cand, pref)

    tau = lax.fori_loop(0, 32, bit_step,
                        jnp.zeros((1, Bb, 1), jnp.uint32))

    gt = key > tau
    eq = key == tau
    m = k - jnp.sum(gt.astype(jnp.int32), axis=(0, 2), keepdims=True)

    idx = (lax.broadcasted_iota(jnp.int32, (NBn, Bb, BNn), 0) * BNn
           + lax.broadcasted_iota(jnp.int32, (NBn, Bb, BNn), 2))

    def cut_step(i, pref):
        cand = pref + (jnp.int32(1) << (14 - i))
        cnt = jnp.sum((eq & (idx < cand)).astype(jnp.int32), axis=(0, 2),
                      keepdims=True)
        return jnp.where(cnt <= m, cand, pref)

    cut = lax.fori_loop(0, 15, cut_step, jnp.zeros((1, Bb, 1), jnp.int32))

    sel = gt | (eq & (idx < cut))
    w_ref[...] = jnp.where(sel, score, 0.0) * (1.0 / k)


def _tc_paccum_body(w_ref, xl_ref, xr_ref, pl_ref, pr_ref):
    i = pl.program_id(0)

    @pl.when(i == 0)
    def _():
        pl_ref[...] = jnp.zeros_like(pl_ref)
        pr_ref[...] = jnp.zeros_like(pr_ref)

    dg = functools.partial(lax.dot_general,
                           preferred_element_type=jnp.float32)
    w = w_ref[0]
    pl_ref[...] += dg(w, xl_ref[...], (((1,), (1,)), ((0,), (0,))))
    pr_ref[...] += dg(w, xr_ref[...], (((1,), (1,)), ((0,), (0,))))


def _tc_mlp_body(half, pl_ref, pr_ref, w1_ref, b1_ref, w2_ref, b2_ref, o_ref):
    dot = functools.partial(jnp.dot, preferred_element_type=jnp.float32)
    h = jnp.maximum(dot(pl_ref[...], w1_ref[0:half, :])
                    + dot(pr_ref[...], w1_ref[half:, :])
                    + b1_ref[...][None, :], 0.0)
    o_ref[...] = dot(h, w2_ref[...]) + b2_ref[...][None, :]


def _tc_pool(x3l, x3r, pool_w, W1, b1, W2, b2):
    Bb, Nn, half = x3l.shape
    k = -(-Nn // 2)
    D = 2 * half
    MH = W1.shape[1]
    C = W2.shape[1]
    BN = 2000
    NB = Nn // BN

    score = pl.pallas_call(
        functools.partial(_tc_score_body, half),
        grid=(NB,),
        in_specs=[
            pl.BlockSpec((Bb, BN, half), lambda i: (0, i, 0)),
            pl.BlockSpec((Bb, BN, half), lambda i: (0, i, 0)),
            pl.BlockSpec((D,), lambda i: (0,)),
        ],
        out_specs=pl.BlockSpec((1, Bb, BN), lambda i: (i, 0, 0)),
        out_shape=jax.ShapeDtypeStruct((NB, Bb, BN), jnp.float32),
    )(x3l, x3r, pool_w)

    w = pl.pallas_call(
        functools.partial(_tc_select_body, k),
        out_shape=jax.ShapeDtypeStruct((NB, Bb, BN), jnp.float32),
    )(score)

    pooled_l, pooled_r = pl.pallas_call(
        _tc_paccum_body,
        grid=(NB,),
        in_specs=[
            pl.BlockSpec((1, Bb, BN), lambda i: (i, 0, 0)),
            pl.BlockSpec((Bb, BN, half), lambda i: (0, i, 0)),
            pl.BlockSpec((Bb, BN, half), lambda i: (0, i, 0)),
        ],
        out_specs=[pl.BlockSpec((Bb, half), lambda i: (0, 0)),
                   pl.BlockSpec((Bb, half), lambda i: (0, 0))],
        out_shape=[jax.ShapeDtypeStruct((Bb, half), jnp.float32),
                   jax.ShapeDtypeStruct((Bb, half), jnp.float32)],
    )(w, x3l, x3r)

    return pl.pallas_call(
        functools.partial(_tc_mlp_body, half),
        out_shape=jax.ShapeDtypeStruct((Bb, C), jnp.float32),
    )(pooled_l, pooled_r, W1, b1, W2, b2)


def kernel(freq, edge_index, edge_weight, Wl0, bl0, Wr0, Wl1, bl1, Wr1,
           Wl2, bl2, Wr2, pool_w, W1, b1, W2, b2):
    B, N, D = freq.shape
    E = edge_index.shape[2]
    half = D // 2

    x0 = freq.reshape(B * N, D)
    x0l = x0[:, 0:half]
    x0r = x0[:, half:]
    offs = (jnp.arange(B, dtype=edge_index.dtype) * N)[:, None]
    n_sub = 16
    rpw = (N // n_sub) // 8 * 8

    srcg_be = edge_index[:, 0, :] + offs
    dstl_be = edge_index[:, 1, :]
    key = ((dstl_be.astype(jnp.uint32) << 18)
           | jnp.arange(E, dtype=jnp.uint32)[None, :])
    ks = jnp.sort(key, axis=1)
    e_s = (ks & jnp.uint32(0x3FFFF)).astype(jnp.int32)
    d_s = (ks >> 18).astype(jnp.int32)
    s_s = jnp.take_along_axis(srcg_be, e_s, axis=1)

    cuts = jnp.arange(n_sub, dtype=jnp.int32) * rpw
    bounds = jax.vmap(lambda row: jnp.searchsorted(row, cuts))(d_s)
    ends = jnp.concatenate(
        [bounds[:, 1:], jnp.full((B, 1), E, jnp.int32)], axis=1)

    capc = 140
    caps = capc * _G
    j = jnp.arange(caps, dtype=jnp.int32)
    pos = bounds[:, :, None] + j[None, None, :]
    valid = pos < ends[:, :, None]
    posc = jnp.minimum(pos, E - 1).reshape(B, -1)
    gsrc = jnp.take_along_axis(s_s, posc, axis=1).reshape(B, n_sub, caps)
    gdst = jnp.take_along_axis(d_s, posc, axis=1).reshape(B, n_sub, caps)
    spread_src = jnp.broadcast_to((j % (B * N))[None, None, :],
                                  (B, n_sub, caps))
    pad_dst = jnp.broadcast_to((N + (j % 8))[None, None, :],
                               (B, n_sub, caps))
    srcg = jnp.where(valid, gsrc, spread_src).reshape(B * n_sub, capc, _G)
    dstl = jnp.where(valid, gdst, pad_dst).reshape(B * n_sub, capc, _G)

    agg_cnt = _make_sc_agg(True, B, N, E, D, capc)
    agg_only = _make_sc_agg(False, B, N, E, D, capc)

    a0l, a0r, cnt = agg_cnt(x0l, x0r, srcg, dstl)
    x1l, x1r = _tc_layer(a0l, a0r, cnt, x0l, x0r, Wl0, Wr0, bl0, relu=True)
    a1l, a1r = agg_only(x1l, x1r, srcg, dstl)
    x2l, x2r = _tc_layer(a1l, a1r, cnt, x1l, x1r, Wl1, Wr1, bl1, relu=True)
    a2l, a2r = agg_only(x2l, x2r, srcg, dstl)
    x3l, x3r = _tc_layer(a2l, a2r, cnt, x2l, x2r, Wl2, Wr2, bl2, relu=False)

    return _tc_pool(x3l.reshape(B, N, half), x3r.reshape(B, N, half),
                    pool_w, W1, b1, W2, b2)
